# baseline (speedup 1.0000x reference)
import functools

import jax
import jax.numpy as jnp
from jax import lax
from jax.experimental import pallas as pl
from jax.experimental.pallas import tpu as pltpu
from jax.experimental.pallas import tpu_sc as plsc

N = 10000
E = 320000
D = 128
NC = 2
NS = 16
NW = NC * NS
EW = E // NW
C = 80
G = EW // C
BL = 128
NBLK = E // BL
NBW = NBLK // NW
NTAIL = NBLK - NBW * NW
HR = 80
HW_ = 8
NPAD = 10240
STRIPE = NPAD // NS

_mesh = plsc.VectorSubcoreMesh(core_axis_name="c", subcore_axis_name="s")


@functools.partial(
    pl.kernel,
    out_type=[
        jax.ShapeDtypeStruct((E,), jnp.int32),
        jax.ShapeDtypeStruct((E,), jnp.int32),
        jax.ShapeDtypeStruct((NC, HR, D), jnp.float32),
    ],
    mesh=_mesh,
    scratch_types=[
        pltpu.VMEM((2, NBW * BL), jnp.int32),
        pltpu.VMEM((2, BL), jnp.int32),
        pltpu.VMEM((HR, D), jnp.float32),
        pltpu.VMEM((HR,), jnp.int32),
        pltpu.VMEM((HW_, D), jnp.float32),
        pltpu.VMEM_SHARED((HR, D), jnp.float32),
        pltpu.SemaphoreType.DMA,
    ],
    compiler_params=pltpu.CompilerParams(needs_layout_passes=False),
)
def _deg_kernel(
    ei_hbm, row_out, col_out, deg_out, eb, tb, hist, rowids, stage, deg_sh, sem
):
    c = lax.axis_index("c")
    s = lax.axis_index("s")
    w = c * NS + s
    base = w * NBW * BL

    zero16 = jnp.zeros((16,), jnp.float32)

    def _zero_hist(i, carry):
        for j in range(D // 16):
            hist[i, pl.ds(j * 16, 16)] = zero16
        return carry

    lax.fori_loop(0, HR, _zero_hist, 0)
    for i in range(HW_):
        for j in range(D // 16):
            stage[i, pl.ds(j * 16, 16)] = zero16
    for k in range(HR // 16):
        rowids[pl.ds(k * 16, 16)] = lax.iota(jnp.int32, 16) + (16 * k)

    @pl.when(s < HR // HW_)
    def _():
        pltpu.sync_copy(stage, deg_sh.at[pl.ds(s * HW_, HW_)])

    pltpu.sync_copy(ei_hbm.at[:, pl.ds(base, NBW * BL)], eb)
    drow = pltpu.async_copy(eb.at[0], row_out.at[pl.ds(base, NBW * BL)], sem)
    dcol = pltpu.async_copy(eb.at[1], col_out.at[pl.ds(base, NBW * BL)], sem)

    @pl.when(w < NTAIL)
    def _():
        tbase = (NBW * NW + w) * BL
        pltpu.sync_copy(ei_hbm.at[:, pl.ds(tbase, BL)], tb)
        pltpu.sync_copy(tb.at[0], row_out.at[pl.ds(tbase, BL)])
        pltpu.sync_copy(tb.at[1], col_out.at[pl.ds(tbase, BL)])

    def _hist_vec(v):
        cnt, lastm = plsc.scan_count(v)
        hi = lax.shift_right_logical(v, 7)
        lo = lax.bitwise_and(v, 127)
        plsc.addupdate_scatter(
            hist, [hi, lo], cnt.astype(jnp.float32), mask=lastm
        )

    def _hist_step(i, carry):
        for j in range(BL // 16):
            _hist_vec(eb[1, pl.ds(i * BL + j * 16, 16)])
        return carry

    lax.fori_loop(0, NBW, _hist_step, 0)

    @pl.when(w < NTAIL)
    def _():
        for j in range(BL // 16):
            _hist_vec(tb[1, pl.ds(j * 16, 16)])

    drow.wait()
    dcol.wait()
    plsc.subcore_barrier()

    pltpu.sync_copy(hist, deg_sh.at[rowids], add=True)
    plsc.subcore_barrier()

    @pl.when(s < HR // HW_)
    def _():
        pltpu.sync_copy(deg_sh.at[pl.ds(s * HW_, HW_)], deg_out.at[c, pl.ds(s * HW_, HW_)])


@functools.partial(
    pl.kernel,
    out_type=jax.ShapeDtypeStruct((NC, NPAD, D), jnp.float32),
    mesh=_mesh,
    scratch_types=[
        pltpu.VMEM((4, C), jnp.int32),
        pltpu.VMEM((4, C), jnp.int32),
        pltpu.VMEM((3, C, D), jnp.float32),
        pltpu.VMEM_SHARED((NPAD, D), jnp.float32),
        pltpu.SemaphoreType.DMA,
        pltpu.SemaphoreType.DMA,
        pltpu.SemaphoreType.DMA,
    ],
)
def _agg_kernel(
    row_hbm, col_hbm, z_hbm, agg_out, rowi, coli, buf, acc, isem, gsem, ssem
):
    c = lax.axis_index("c")
    s = lax.axis_index("s")
    w = c * NS + s

    zero16 = jnp.zeros((16,), jnp.float32)

    def _zero_buf(i, carry):
        for j in range(D // 16):
            buf[0, i, pl.ds(j * 16, 16)] = zero16
        return carry

    lax.fori_loop(0, C, _zero_buf, 0)
    zdescs = [
        pltpu.async_copy(
            buf.at[0], acc.at[pl.ds(s * STRIPE + k * C, C)], ssem
        )
        for k in range(STRIPE // C)
    ]

    idx_descs = {}
    gat_descs = {}
    sca_descs = {}

    def _load_idx(g):
        base = w * EW + g * C
        idx_descs[g] = (
            pltpu.async_copy(row_hbm.at[pl.ds(base, C)], rowi.at[g % 4], isem),
            pltpu.async_copy(col_hbm.at[pl.ds(base, C)], coli.at[g % 4], isem),
        )

    def _wait_idx(g):
        for dd in idx_descs.pop(g):
            dd.wait()

    def _gather(g):
        gat_descs[g] = pltpu.async_copy(
            z_hbm.at[rowi.at[g % 4]], buf.at[g % 3], gsem
        )

    def _scatter(g):
        sca_descs[g] = pltpu.async_copy(
            buf.at[g % 3], acc.at[coli.at[g % 4]], ssem, add=True
        )

    for g in range(min(3, G)):
        _load_idx(g)
    for dd in zdescs:
        dd.wait()
    _wait_idx(0)
    plsc.subcore_barrier()
    _gather(0)
    if G > 1:
        _wait_idx(1)
        _gather(1)

    for g in range(G):
        gat_descs.pop(g).wait()
        _scatter(g)
        if g - 1 >= 0:
            sca_descs.pop(g - 1).wait()
        if g + 2 < G:
            _wait_idx(g + 2)
            _gather(g + 2)
        if g + 3 < G:
            _load_idx(g + 3)
    sca_descs.pop(G - 1).wait()
    plsc.subcore_barrier()

    pltpu.sync_copy(
        acc.at[pl.ds(s * STRIPE, STRIPE)],
        agg_out.at[c, pl.ds(s * STRIPE, STRIPE)],
    )


_R = 10240
_NRB = NPAD // _R
_HB = _R // D


def _dinv_col(dp_ref):
    deg = dp_ref[0] + dp_ref[1] + 1.0
    dt = lax.rsqrt(deg).T
    return jnp.concatenate(
        [jnp.broadcast_to(dt[:, a : a + 1], (D, D)) for a in range(_HB)],
        axis=0,
    )


def _scale_body(x_ref, dp_ref, z_ref):
    z_ref[...] = x_ref[...] * _dinv_col(dp_ref)


_scale_call = pl.pallas_call(
    _scale_body,
    grid=(_NRB,),
    in_specs=[
        pl.BlockSpec((_R, D), lambda i: (i, 0)),
        pl.BlockSpec((NC, _HB, D), lambda i: (0, i, 0)),
    ],
    out_specs=pl.BlockSpec((_R, D), lambda i: (i, 0)),
    out_shape=jax.ShapeDtypeStruct((NPAD, D), jnp.float32),
)


def _final_body(a_ref, z_ref, dp_ref, w_ref, b_ref, o_ref):
    sm = a_ref[0] + a_ref[1] + z_ref[...]
    m = lax.dot_general(
        sm, w_ref[...], (((1,), (1,)), ((), ())),
        preferred_element_type=jnp.float32,
    )
    o_ref[...] = jnp.maximum(m * _dinv_col(dp_ref) + b_ref[...], 0.0)


_final_call = pl.pallas_call(
    _final_body,
    grid=(_NRB,),
    in_specs=[
        pl.BlockSpec((NC, _R, D), lambda i: (0, i, 0)),
        pl.BlockSpec((_R, D), lambda i: (i, 0)),
        pl.BlockSpec((NC, _HB, D), lambda i: (0, i, 0)),
        pl.BlockSpec((D, D), lambda i: (0, 0)),
        pl.BlockSpec((1, D), lambda i: (0, 0)),
    ],
    out_specs=pl.BlockSpec((_R, D), lambda i: (i, 0)),
    out_shape=jax.ShapeDtypeStruct((N, D), jnp.float32),
)


def kernel(x, edge_index, W, b):
    xp = jnp.pad(x, ((0, NPAD - N), (0, 0)))

    row, col, degp = _deg_kernel(edge_index)
    z = _scale_call(xp, degp)
    aggp = _agg_kernel(row, col, z)
    h = _final_call(aggp, z, degp, W, b[None, :])
    return h.reshape(-1)


# speedup vs baseline: 1.0194x; 1.0194x over previous
import functools

import jax
import jax.numpy as jnp
from jax import lax
from jax.experimental import pallas as pl
from jax.experimental.pallas import tpu as pltpu
from jax.experimental.pallas import tpu_sc as plsc

N = 10000
E = 320000
D = 128
NC = 2
NS = 16
NW = NC * NS
EW = E // NW
C = 80
G = EW // C
BL = 128
NBLK = E // BL
NBW = NBLK // NW
NTAIL = NBLK - NBW * NW
HR = 80
HW_ = 8
NPAD = 10240
STRIPE = NPAD // NS

_mesh = plsc.VectorSubcoreMesh(core_axis_name="c", subcore_axis_name="s")


@functools.partial(
    pl.kernel,
    out_type=[
        jax.ShapeDtypeStruct((E,), jnp.int32),
        jax.ShapeDtypeStruct((E,), jnp.int32),
        jax.ShapeDtypeStruct((NC, HR, D), jnp.float32),
    ],
    mesh=_mesh,
    scratch_types=[
        pltpu.VMEM((2, NBW * BL), jnp.int32),
        pltpu.VMEM((2, BL), jnp.int32),
        pltpu.VMEM((HR, D), jnp.float32),
        pltpu.VMEM((HR,), jnp.int32),
        pltpu.VMEM((HW_, D), jnp.float32),
        pltpu.VMEM_SHARED((HR, D), jnp.float32),
        pltpu.SemaphoreType.DMA,
    ],
    compiler_params=pltpu.CompilerParams(needs_layout_passes=False),
)
def _deg_kernel(
    ei_hbm, row_out, col_out, deg_out, eb, tb, hist, rowids, stage, deg_sh, sem
):
    c = lax.axis_index("c")
    s = lax.axis_index("s")
    w = c * NS + s
    base = w * NBW * BL

    zero16 = jnp.zeros((16,), jnp.float32)

    def _zero_hist(i, carry):
        for j in range(D // 16):
            hist[i, pl.ds(j * 16, 16)] = zero16
        return carry

    lax.fori_loop(0, HR, _zero_hist, 0)
    for i in range(HW_):
        for j in range(D // 16):
            stage[i, pl.ds(j * 16, 16)] = zero16
    for k in range(HR // 16):
        rowids[pl.ds(k * 16, 16)] = lax.iota(jnp.int32, 16) + (16 * k)

    @pl.when(s < HR // HW_)
    def _():
        pltpu.sync_copy(stage, deg_sh.at[pl.ds(s * HW_, HW_)])

    pltpu.sync_copy(ei_hbm.at[:, pl.ds(base, NBW * BL)], eb)
    drow = pltpu.async_copy(eb.at[0], row_out.at[pl.ds(base, NBW * BL)], sem)
    dcol = pltpu.async_copy(eb.at[1], col_out.at[pl.ds(base, NBW * BL)], sem)

    @pl.when(w < NTAIL)
    def _():
        tbase = (NBW * NW + w) * BL
        pltpu.sync_copy(ei_hbm.at[:, pl.ds(tbase, BL)], tb)
        pltpu.sync_copy(tb.at[0], row_out.at[pl.ds(tbase, BL)])
        pltpu.sync_copy(tb.at[1], col_out.at[pl.ds(tbase, BL)])

    def _hist_vec(v):
        cnt, lastm = plsc.scan_count(v)
        hi = lax.shift_right_logical(v, 7)
        lo = lax.bitwise_and(v, 127)
        plsc.addupdate_scatter(
            hist, [hi, lo], cnt.astype(jnp.float32), mask=lastm
        )

    def _hist_step(i, carry):
        for j in range(BL // 16):
            _hist_vec(eb[1, pl.ds(i * BL + j * 16, 16)])
        return carry

    lax.fori_loop(0, NBW, _hist_step, 0)

    @pl.when(w < NTAIL)
    def _():
        for j in range(BL // 16):
            _hist_vec(tb[1, pl.ds(j * 16, 16)])

    drow.wait()
    dcol.wait()
    plsc.subcore_barrier()

    pltpu.sync_copy(hist, deg_sh.at[rowids], add=True)
    plsc.subcore_barrier()

    @pl.when(s < HR // HW_)
    def _():
        pltpu.sync_copy(deg_sh.at[pl.ds(s * HW_, HW_)], deg_out.at[c, pl.ds(s * HW_, HW_)])


@functools.partial(
    pl.kernel,
    out_type=jax.ShapeDtypeStruct((NC, NPAD, D), jnp.float32),
    mesh=_mesh,
    scratch_types=[
        pltpu.VMEM((4, C), jnp.int32),
        pltpu.VMEM((4, C), jnp.int32),
        pltpu.VMEM((3, C, D), jnp.float32),
        pltpu.VMEM_SHARED((NPAD, D), jnp.float32),
        pltpu.SemaphoreType.DMA,
        pltpu.SemaphoreType.DMA,
        pltpu.SemaphoreType.DMA,
    ],
)
def _agg_kernel(
    row_hbm, col_hbm, z_hbm, agg_out, rowi, coli, buf, acc, isem, gsem, ssem
):
    c = lax.axis_index("c")
    s = lax.axis_index("s")
    w = c * NS + s

    zero16 = jnp.zeros((16,), jnp.float32)

    def _zero_buf(i, carry):
        for j in range(D // 16):
            buf[0, i, pl.ds(j * 16, 16)] = zero16
        return carry

    lax.fori_loop(0, C, _zero_buf, 0)
    zdescs = [
        pltpu.async_copy(
            buf.at[0], acc.at[pl.ds(s * STRIPE + k * C, C)], ssem
        )
        for k in range(STRIPE // C)
    ]

    idx_descs = {}
    gat_descs = {}
    sca_descs = {}

    def _load_idx(g):
        base = w * EW + g * C
        idx_descs[g] = (
            pltpu.async_copy(row_hbm.at[pl.ds(base, C)], rowi.at[g % 4], isem),
            pltpu.async_copy(col_hbm.at[pl.ds(base, C)], coli.at[g % 4], isem),
        )

    def _wait_idx(g):
        for dd in idx_descs.pop(g):
            dd.wait()

    def _gather(g):
        gat_descs[g] = pltpu.async_copy(
            z_hbm.at[rowi.at[g % 4]], buf.at[g % 3], gsem
        )

    def _scatter(g):
        sca_descs[g] = pltpu.async_copy(
            buf.at[g % 3], acc.at[coli.at[g % 4]], ssem, add=True
        )

    for g in range(min(3, G)):
        _load_idx(g)
    for dd in zdescs:
        dd.wait()
    _wait_idx(0)
    plsc.subcore_barrier()
    _gather(0)
    if G > 1:
        _wait_idx(1)
        _gather(1)

    for g in range(G):
        gat_descs.pop(g).wait()
        _scatter(g)
        if g - 1 >= 0:
            sca_descs.pop(g - 1).wait()
        if g + 2 < G:
            _wait_idx(g + 2)
            _gather(g + 2)
        if g + 3 < G:
            _load_idx(g + 3)
    sca_descs.pop(G - 1).wait()
    plsc.subcore_barrier()

    pltpu.sync_copy(
        acc.at[pl.ds(s * STRIPE, STRIPE)],
        agg_out.at[c, pl.ds(s * STRIPE, STRIPE)],
    )


_R = 5120
_NRB = NPAD // _R
_HB = _R // D


def _dinv_col(dp_ref):
    deg = dp_ref[0] + dp_ref[1] + 1.0
    dt = lax.rsqrt(deg).T
    return jnp.concatenate(
        [jnp.broadcast_to(dt[:, a : a + 1], (D, D)) for a in range(_HB)],
        axis=0,
    )


def _scale_body(x_ref, dp_ref, z_ref):
    z_ref[...] = x_ref[...] * _dinv_col(dp_ref)


_scale_call = pl.pallas_call(
    _scale_body,
    grid=(_NRB,),
    in_specs=[
        pl.BlockSpec((_R, D), lambda i: (i, 0)),
        pl.BlockSpec((NC, _HB, D), lambda i: (0, i, 0)),
    ],
    out_specs=pl.BlockSpec((_R, D), lambda i: (i, 0)),
    out_shape=jax.ShapeDtypeStruct((NPAD, D), jnp.float32),
)


def _final_body(a_ref, z_ref, dp_ref, w_ref, b_ref, o_ref):
    sm = a_ref[0] + a_ref[1] + z_ref[...]
    m = lax.dot_general(
        sm, w_ref[...], (((1,), (1,)), ((), ())),
        preferred_element_type=jnp.float32,
    )
    o_ref[...] = jnp.maximum(m * _dinv_col(dp_ref) + b_ref[...], 0.0)


_final_call = pl.pallas_call(
    _final_body,
    grid=(_NRB,),
    in_specs=[
        pl.BlockSpec((NC, _R, D), lambda i: (0, i, 0)),
        pl.BlockSpec((_R, D), lambda i: (i, 0)),
        pl.BlockSpec((NC, _HB, D), lambda i: (0, i, 0)),
        pl.BlockSpec((D, D), lambda i: (0, 0)),
        pl.BlockSpec((1, D), lambda i: (0, 0)),
    ],
    out_specs=pl.BlockSpec((_R, D), lambda i: (i, 0)),
    out_shape=jax.ShapeDtypeStruct((N, D), jnp.float32),
)


def kernel(x, edge_index, W, b):
    xp = jnp.pad(x, ((0, NPAD - N), (0, 0)))

    row, col, degp = _deg_kernel(edge_index)
    z = _scale_call(xp, degp)
    aggp = _agg_kernel(row, col, z)
    h = _final_call(aggp, z, degp, W, b[None, :])
    return h.reshape(-1)
